# baseline (speedup 1.0000x reference)
import functools

import jax
import jax.numpy as jnp
import numpy as np
from jax import lax
from jax.experimental import pallas as pl
from jax.experimental.pallas import tpu as pltpu
from jax.experimental.pallas import tpu_sc as plsc

N_NODES = 10000
N_EDGES = 320000
DIM = 128

NPAD = 10240
PAD_IDX = 10200
NC = 2
NS = 16
NT = NC * NS
CH = 128
NCH = 80
EPT = NCH * CH
EPAD = NT * EPT
ROWS_PER_TILE = NPAD // NS

_f32 = jnp.float32
_i32 = jnp.int32

_PERM = np.arange(DIM).reshape(DIM // 32, 2, 16).transpose(0, 2, 1).reshape(
    DIM)


def _prep_body(h_ref, w_ref, wh_ref):
    hb = h_ref[...]
    wm = w_ref[...]
    wh_ref[...] = lax.dot_general(hb, wm, (((1,), (1,)), ((), ())),
                                  preferred_element_type=_f32)


def _prep(h, W):
    blk = 1000
    grid = N_NODES // blk
    return pl.pallas_call(
        _prep_body,
        grid=(grid,),
        in_specs=[
            pl.BlockSpec((blk, DIM), lambda i: (i, 0)),
            pl.BlockSpec((DIM, DIM), lambda i: (0, 0)),
        ],
        out_specs=pl.BlockSpec((blk, DIM), lambda i: (i, 0)),
        out_shape=jax.ShapeDtypeStruct((N_NODES, DIM), _f32),
    )(h, W)


def _sdprep_body(h_ref, w_ref, a_ref, sd_ref):
    aw = lax.dot_general(a_ref[...], w_ref[...], (((1,), (0,)), ((), ())),
                         preferred_element_type=_f32)
    sd = lax.dot_general(aw, h_ref[...], (((1,), (1,)), ((), ())),
                         preferred_element_type=_f32)
    sd_ref[...] = jnp.concatenate(
        [sd, jnp.zeros((2, NPAD - N_NODES), _f32)], axis=1)


def _sdprep(h, W, a12):
    return pl.pallas_call(
        _sdprep_body,
        out_shape=jax.ShapeDtypeStruct((2, NPAD), _f32),
    )(h, W, a12)


def _pass1_body(sd_hbm, srcp_hbm, dstp_hbm,
                aexp_hbm, hist_hbm,
                s_v, d_v, src_v, dst_v, aexp_v, zero_v, hist_sh, ldsem):
    cid = lax.axis_index("c")
    sid = lax.axis_index("s")
    wid = cid * NS + sid

    c1 = pltpu.async_copy(sd_hbm.at[0], s_v, ldsem)
    c2 = pltpu.async_copy(sd_hbm.at[1], d_v, ldsem)
    c4 = pltpu.async_copy(srcp_hbm.at[wid], src_v, ldsem)
    c5 = pltpu.async_copy(dstp_hbm.at[wid], dst_v, ldsem)

    for k in range(ROWS_PER_TILE // 16):
        zero_v[pl.ds(k * 16, 16)] = jnp.zeros((16,), _f32)
    pltpu.sync_copy(zero_v, hist_sh.at[pl.ds(sid * ROWS_PER_TILE,
                                             ROWS_PER_TILE)])
    for c in (c1, c2, c4, c5):
        c.wait()
    plsc.subcore_barrier()

    def _maxstep(i, carry):
        ms, md = carry
        sl = pl.ds(i * 16, 16)
        return (jnp.maximum(ms, s_v[sl]), jnp.maximum(md, d_v[sl]))

    ms0 = jnp.full((16,), -jnp.inf, _f32)
    ms, md = lax.fori_loop(0, NPAD // 16, _maxstep, (ms0, ms0),
                           unroll=2)
    mvec = jnp.full((16,), jnp.maximum(
        lax.reduce_max(ms, (0,)) + lax.reduce_max(md, (0,)), 0.0), _f32)

    @pl.loop(0, NCH, unroll=2)
    def _compute(r):
        for g in range(8):
            sl = pl.ds(g * 16, 16)
            si = src_v[r, sl]
            di = dst_v[r, sl]
            sg = plsc.load_gather(s_v, [si])
            dg = plsc.load_gather(d_v, [di])
            al = sg + dg
            al = jnp.where(al > 0, al, al * 0.2)
            aexp_v[r, sl] = jnp.exp(al - mvec)

    @pl.loop(0, NCH)
    def _scatter(r):
        pltpu.async_copy(aexp_v.at[r], hist_sh.at[dst_v.at[r]], ldsem,
                         add=True)

    @pl.loop(0, NCH)
    def _drain(r):
        pltpu.make_async_copy(aexp_v.at[r], hist_sh.at[dst_v.at[r]],
                              ldsem).wait()

    plsc.subcore_barrier()
    pltpu.sync_copy(aexp_v, aexp_hbm.at[wid])

    @pl.when(sid == 0)
    def _dump():
        pltpu.sync_copy(hist_sh, s_v)
        pltpu.sync_copy(s_v, hist_hbm.at[cid])


def _pass1(sd, srcp, dstp):
    mesh = plsc.VectorSubcoreMesh(core_axis_name="c", subcore_axis_name="s")
    fn = pl.kernel(
        _pass1_body,
        out_type=(
            jax.ShapeDtypeStruct((NT, NCH, CH), _f32),
            jax.ShapeDtypeStruct((NC, NPAD), _f32),
        ),
        mesh=mesh,
        scratch_types=[
            pltpu.VMEM((NPAD,), _f32),
            pltpu.VMEM((NPAD,), _f32),
            pltpu.VMEM((NCH, CH), _i32),
            pltpu.VMEM((NCH, CH), _i32),
            pltpu.VMEM((NCH, CH), _f32),
            pltpu.VMEM((ROWS_PER_TILE,), _f32),
            pltpu.VMEM_SHARED((NPAD,), _f32),
            pltpu.SemaphoreType.DMA,
        ],
        compiler_params=pltpu.CompilerParams(needs_layout_passes=False),
    )
    return fn(sd, srcp, dstp)


def _pass2_body(srcp_hbm, dstp_hbm, aexp_hbm, hist_hbm, whbf_hbm,
                an_hbm, outp_hbm,
                src0, src1, dst0, dst1, ae0, ae1, an0, an1,
                asum_v, tmp_v, rbf0, rbf1, rows_f, acc_sh,
                gsem0, gsem1, psem0, psem1, asem0, asem1):
    cid = lax.axis_index("c")
    sid = lax.axis_index("s")
    wid = cid * NS + sid

    bufs = [(src0, dst0, ae0, an0, rbf0, gsem0, psem0, asem0),
            (src1, dst1, ae1, an1, rbf1, gsem1, psem1, asem1)]

    pltpu.sync_copy(hist_hbm.at[0], asum_v)
    for k in range(8):
        csl = pl.ds(k * (NPAD // 8), NPAD // 8)
        pltpu.sync_copy(hist_hbm.at[1, csl], tmp_v)

        @pl.loop(0, NPAD // 8 // 16)
        def _acc(i):
            sl = pl.ds(k * (NPAD // 8) + i * 16, 16)
            tsl = pl.ds(i * 16, 16)
            asum_v[sl] = asum_v[sl] + tmp_v[tsl]

    @pl.loop(0, CH)
    def _zero(r):
        for g in range(8):
            rows_f[r, pl.ds(g * 16, 16)] = jnp.zeros((16,), _f32)

    for k in range(ROWS_PER_TILE // CH):
        pltpu.sync_copy(rows_f,
                        acc_sh.at[pl.ds(sid * ROWS_PER_TILE + k * CH, CH)])
    plsc.subcore_barrier()

    pltpu.sync_copy(srcp_hbm.at[wid, 0], src0)
    pltpu.sync_copy(dstp_hbm.at[wid, 0], dst0)
    pltpu.sync_copy(aexp_hbm.at[wid, 0], ae0)
    pltpu.async_copy(whbf_hbm.at[src0], rbf0, gsem0)
    pltpu.async_copy(srcp_hbm.at[wid, 1], src1, psem1)
    pltpu.async_copy(dstp_hbm.at[wid, 1], dst1, psem1)
    pltpu.async_copy(aexp_hbm.at[wid, 1], ae1, psem1)

    @pl.loop(0, NCH, step=2)
    def _chunk(r):
        for b in range(2):
            rr = r + b
            src_b, dst_b, ae_b, an_b, rbf_b, gsem_b, psem_b, asem_b = bufs[b]
            src_o, dst_o, ae_o, an_o, rbf_o, gsem_o, psem_o, asem_o = \
                bufs[1 - b]

            @pl.when(rr + 1 < NCH)
            def _launch_next():
                pltpu.make_async_copy(srcp_hbm.at[wid, rr + 1], src_o,
                                      psem_o).wait()
                pltpu.make_async_copy(dstp_hbm.at[wid, rr + 1], dst_o,
                                      psem_o).wait()
                pltpu.make_async_copy(aexp_hbm.at[wid, rr + 1], ae_o,
                                      psem_o).wait()
                pltpu.async_copy(whbf_hbm.at[src_o], rbf_o, gsem_o)

            @pl.when(rr >= 2)
            def _drain_an():
                pltpu.make_async_copy(an_b, an_hbm.at[wid, rr], asem_b).wait()

            for g in range(8):
                sl = pl.ds(g * 16, 16)
                asg = plsc.load_gather(asum_v, [dst_b[sl]])
                an_b[sl] = ae_b[sl] / (asg + 1e-9)
            pltpu.async_copy(an_b, an_hbm.at[wid, rr], asem_b)

            pltpu.make_async_copy(whbf_hbm.at[src_b], rbf_b, gsem_b).wait()

            @pl.loop(0, CH, unroll=2)
            def _scale(e):
                aeb = plsc.load_gather(ae_b, [jnp.full((16,), e, _i32)])
                for g in range(4):
                    x = plsc.bitcast(rbf_b[e, pl.ds(g * 16, 16)],
                                     jnp.bfloat16)
                    lo, hi = plsc.unpack(x, format=plsc.PackFormat.INTERLEAVED)
                    rows_f[e, pl.ds(g * 32, 16)] = lo * aeb
                    rows_f[e, pl.ds(g * 32 + 16, 16)] = hi * aeb

            pltpu.sync_copy(rows_f, acc_sh.at[dst_b], add=True)

            @pl.when(rr + 2 < NCH)
            def _prefetch():
                pltpu.async_copy(srcp_hbm.at[wid, rr + 2], src_b, psem_b)
                pltpu.async_copy(dstp_hbm.at[wid, rr + 2], dst_b, psem_b)
                pltpu.async_copy(aexp_hbm.at[wid, rr + 2], ae_b, psem_b)

    pltpu.make_async_copy(an0, an_hbm.at[wid, NCH - 2], asem0).wait()
    pltpu.make_async_copy(an1, an_hbm.at[wid, NCH - 1], asem1).wait()
    plsc.subcore_barrier()

    half = CH // 2
    nko = ROWS_PER_TILE // half
    for k in range(nko):
        hsl = pl.ds((k % 2) * half, half)
        wsem = gsem0 if k % 2 == 0 else gsem1
        base = sid * ROWS_PER_TILE + k * half
        if k >= 2:
            pltpu.make_async_copy(
                rows_f.at[pl.ds(((k - 2) % 2) * half, half)],
                outp_hbm.at[cid, pl.ds(sid * ROWS_PER_TILE
                                       + (k - 2) * half, half)], wsem).wait()
        pltpu.sync_copy(acc_sh.at[pl.ds(base, half)], rows_f.at[hsl])
        pltpu.async_copy(rows_f.at[hsl], outp_hbm.at[cid, pl.ds(base, half)],
                         wsem)
    for k in (nko - 2, nko - 1):
        wsem = gsem0 if k % 2 == 0 else gsem1
        pltpu.make_async_copy(
            rows_f.at[pl.ds((k % 2) * half, half)],
            outp_hbm.at[cid, pl.ds(sid * ROWS_PER_TILE + k * half, half)],
            wsem).wait()


def _pass2(srcp, dstp, aexp, hist, whbf):
    mesh = plsc.VectorSubcoreMesh(core_axis_name="c", subcore_axis_name="s")
    fn = pl.kernel(
        _pass2_body,
        out_type=(
            jax.ShapeDtypeStruct((NT, NCH, CH), _f32),
            jax.ShapeDtypeStruct((NC, NPAD, DIM), _f32),
        ),
        mesh=mesh,
        scratch_types=[
            pltpu.VMEM((CH,), _i32),
            pltpu.VMEM((CH,), _i32),
            pltpu.VMEM((CH,), _i32),
            pltpu.VMEM((CH,), _i32),
            pltpu.VMEM((CH,), _f32),
            pltpu.VMEM((CH,), _f32),
            pltpu.VMEM((CH,), _f32),
            pltpu.VMEM((CH,), _f32),
            pltpu.VMEM((NPAD,), _f32),
            pltpu.VMEM((NPAD // 8,), _f32),
            pltpu.VMEM((CH, DIM // 2), _i32),
            pltpu.VMEM((CH, DIM // 2), _i32),
            pltpu.VMEM((CH, DIM), _f32),
            pltpu.VMEM_SHARED((NPAD, DIM), _f32),
            pltpu.SemaphoreType.DMA,
            pltpu.SemaphoreType.DMA,
            pltpu.SemaphoreType.DMA,
            pltpu.SemaphoreType.DMA,
            pltpu.SemaphoreType.DMA,
            pltpu.SemaphoreType.DMA,
        ],
        compiler_params=pltpu.CompilerParams(needs_layout_passes=False,
                                             use_tc_tiling_on_sc=False),
    )
    return fn(srcp, dstp, aexp, hist, whbf)


def _fin_body(p_ref, a_ref, o_ref):
    x = (p_ref[0] + p_ref[1]) / (a_ref[...] + 1e-9)
    o_ref[...] = jnp.where(x > 0, x, jnp.exp(x) - 1.0)


def _fin(outp, asum2):
    blk = 1000
    grid = N_NODES // blk
    return pl.pallas_call(
        _fin_body,
        grid=(grid,),
        in_specs=[
            pl.BlockSpec((NC, blk, DIM), lambda i: (0, i, 0)),
            pl.BlockSpec((blk, 1), lambda i: (i, 0)),
        ],
        out_specs=pl.BlockSpec((blk, DIM), lambda i: (i, 0)),
        out_shape=jax.ShapeDtypeStruct((N_NODES, DIM), _f32),
    )(outp, asum2)


def kernel(h, edge_index, W, a_w):
    a12 = a_w.reshape(2, DIM)

    rot = jnp.arange(EPAD - N_EDGES, dtype=_i32) % (NPAD - N_NODES)
    srcp = jnp.concatenate([edge_index[0], rot]).reshape(NT, NCH, CH)
    dstp = jnp.concatenate([edge_index[1], N_NODES + rot]).reshape(
        NT, NCH, CH)

    sd = _sdprep(h, W, a12)
    wh = _prep(h, W)
    whbf = lax.bitcast_convert_type(
        wh.astype(jnp.bfloat16)[:, _PERM].reshape(N_NODES, DIM // 2, 2),
        jnp.int32)
    aexp, hist = _pass1(sd, srcp, dstp)
    anorm, outp = _pass2(srcp, dstp, aexp, hist, whbf)
    asum2 = (hist[0, :N_NODES] + hist[1, :N_NODES])[:, None]
    out = _fin(outp, asum2)
    return (out, anorm.reshape(EPAD)[:N_EDGES])


# speedup vs baseline: 1.0001x; 1.0001x over previous
import functools

import jax
import jax.numpy as jnp
import numpy as np
from jax import lax
from jax.experimental import pallas as pl
from jax.experimental.pallas import tpu as pltpu
from jax.experimental.pallas import tpu_sc as plsc

N_NODES = 10000
N_EDGES = 320000
DIM = 128

NPAD = 10240
PAD_IDX = 10200
NC = 2
NS = 16
NT = NC * NS
CH = 128
NCH = 80
EPT = NCH * CH
EPAD = NT * EPT
ROWS_PER_TILE = NPAD // NS

_f32 = jnp.float32
_i32 = jnp.int32

_PERM = np.arange(DIM).reshape(DIM // 32, 2, 16).transpose(0, 2, 1).reshape(
    DIM)


def _prep_body(h_ref, w_ref, wh_ref):
    hb = h_ref[...]
    wm = w_ref[...]
    wh_ref[...] = lax.dot_general(hb, wm, (((1,), (1,)), ((), ())),
                                  preferred_element_type=_f32)


def _prep(h, W):
    blk = 1000
    grid = N_NODES // blk
    return pl.pallas_call(
        _prep_body,
        grid=(grid,),
        in_specs=[
            pl.BlockSpec((blk, DIM), lambda i: (i, 0)),
            pl.BlockSpec((DIM, DIM), lambda i: (0, 0)),
        ],
        out_specs=pl.BlockSpec((blk, DIM), lambda i: (i, 0)),
        out_shape=jax.ShapeDtypeStruct((N_NODES, DIM), _f32),
    )(h, W)


def _sdprep_body(h_ref, w_ref, a_ref, sd_ref):
    aw = lax.dot_general(a_ref[...], w_ref[...], (((1,), (0,)), ((), ())),
                         preferred_element_type=_f32)
    sd = lax.dot_general(aw, h_ref[...], (((1,), (1,)), ((), ())),
                         preferred_element_type=_f32)
    sd_ref[...] = jnp.concatenate(
        [sd, jnp.zeros((2, NPAD - N_NODES), _f32)], axis=1)


def _sdprep(h, W, a12):
    return pl.pallas_call(
        _sdprep_body,
        out_shape=jax.ShapeDtypeStruct((2, NPAD), _f32),
    )(h, W, a12)


def _pass1_body(sd_hbm, srcp_hbm, dstp_hbm,
                aexp_hbm, hist_hbm,
                s_v, d_v, src_v, dst_v, aexp_v, zero_v, hist_sh, ldsem):
    cid = lax.axis_index("c")
    sid = lax.axis_index("s")
    wid = cid * NS + sid

    c1 = pltpu.async_copy(sd_hbm.at[0], s_v, ldsem)
    c2 = pltpu.async_copy(sd_hbm.at[1], d_v, ldsem)
    c4 = pltpu.async_copy(srcp_hbm.at[wid], src_v, ldsem)
    c5 = pltpu.async_copy(dstp_hbm.at[wid], dst_v, ldsem)

    for k in range(ROWS_PER_TILE // 16):
        zero_v[pl.ds(k * 16, 16)] = jnp.zeros((16,), _f32)
    pltpu.sync_copy(zero_v, hist_sh.at[pl.ds(sid * ROWS_PER_TILE,
                                             ROWS_PER_TILE)])
    for c in (c1, c2, c4, c5):
        c.wait()
    plsc.subcore_barrier()

    def _maxstep(i, carry):
        ms, md = carry
        sl = pl.ds(i * 16, 16)
        return (jnp.maximum(ms, s_v[sl]), jnp.maximum(md, d_v[sl]))

    ms0 = jnp.full((16,), -jnp.inf, _f32)
    ms, md = lax.fori_loop(0, NPAD // 16, _maxstep, (ms0, ms0),
                           unroll=2)
    mvec = jnp.full((16,), jnp.maximum(
        lax.reduce_max(ms, (0,)) + lax.reduce_max(md, (0,)), 0.0), _f32)

    @pl.loop(0, NCH, unroll=2)
    def _compute(r):
        for g in range(8):
            sl = pl.ds(g * 16, 16)
            si = src_v[r, sl]
            di = dst_v[r, sl]
            sg = plsc.load_gather(s_v, [si])
            dg = plsc.load_gather(d_v, [di])
            al = sg + dg
            al = jnp.where(al > 0, al, al * 0.2)
            aexp_v[r, sl] = jnp.exp(al - mvec)

    @pl.loop(0, NCH)
    def _scatter(r):
        pltpu.async_copy(aexp_v.at[r], hist_sh.at[dst_v.at[r]], ldsem,
                         add=True)

    @pl.loop(0, NCH)
    def _drain(r):
        pltpu.make_async_copy(aexp_v.at[r], hist_sh.at[dst_v.at[r]],
                              ldsem).wait()

    plsc.subcore_barrier()
    pltpu.sync_copy(aexp_v, aexp_hbm.at[wid])

    @pl.when(sid == 0)
    def _dump():
        pltpu.sync_copy(hist_sh, s_v)
        pltpu.sync_copy(s_v, hist_hbm.at[cid])


def _pass1(sd, srcp, dstp):
    mesh = plsc.VectorSubcoreMesh(core_axis_name="c", subcore_axis_name="s")
    fn = pl.kernel(
        _pass1_body,
        out_type=(
            jax.ShapeDtypeStruct((NT, NCH, CH), _f32),
            jax.ShapeDtypeStruct((NC, NPAD), _f32),
        ),
        mesh=mesh,
        scratch_types=[
            pltpu.VMEM((NPAD,), _f32),
            pltpu.VMEM((NPAD,), _f32),
            pltpu.VMEM((NCH, CH), _i32),
            pltpu.VMEM((NCH, CH), _i32),
            pltpu.VMEM((NCH, CH), _f32),
            pltpu.VMEM((ROWS_PER_TILE,), _f32),
            pltpu.VMEM_SHARED((NPAD,), _f32),
            pltpu.SemaphoreType.DMA,
        ],
        compiler_params=pltpu.CompilerParams(needs_layout_passes=False),
    )
    return fn(sd, srcp, dstp)


def _pass2_body(srcp_hbm, dstp_hbm, aexp_hbm, hist_hbm, whbf_hbm,
                an_hbm, outp_hbm,
                src0, src1, dst0, dst1, ae0, ae1, an0, an1,
                asum_v, tmp_v, rbf0, rbf1, rows_f, acc_sh,
                gsem0, gsem1, psem0, psem1, asem0, asem1):
    cid = lax.axis_index("c")
    sid = lax.axis_index("s")
    wid = cid * NS + sid

    bufs = [(src0, dst0, ae0, an0, rbf0, gsem0, psem0, asem0),
            (src1, dst1, ae1, an1, rbf1, gsem1, psem1, asem1)]

    pltpu.sync_copy(hist_hbm.at[0], asum_v)
    for k in range(8):
        csl = pl.ds(k * (NPAD // 8), NPAD // 8)
        pltpu.sync_copy(hist_hbm.at[1, csl], tmp_v)

        @pl.loop(0, NPAD // 8 // 16)
        def _acc(i):
            sl = pl.ds(k * (NPAD // 8) + i * 16, 16)
            tsl = pl.ds(i * 16, 16)
            asum_v[sl] = asum_v[sl] + tmp_v[tsl]

    @pl.loop(0, CH)
    def _zero(r):
        for g in range(8):
            rows_f[r, pl.ds(g * 16, 16)] = jnp.zeros((16,), _f32)

    for k in range(ROWS_PER_TILE // CH):
        pltpu.sync_copy(rows_f,
                        acc_sh.at[pl.ds(sid * ROWS_PER_TILE + k * CH, CH)])
    plsc.subcore_barrier()

    pltpu.sync_copy(srcp_hbm.at[wid, 0], src0)
    pltpu.sync_copy(dstp_hbm.at[wid, 0], dst0)
    pltpu.sync_copy(aexp_hbm.at[wid, 0], ae0)
    pltpu.async_copy(whbf_hbm.at[src0], rbf0, gsem0)
    pltpu.async_copy(srcp_hbm.at[wid, 1], src1, psem1)
    pltpu.async_copy(dstp_hbm.at[wid, 1], dst1, psem1)
    pltpu.async_copy(aexp_hbm.at[wid, 1], ae1, psem1)

    @pl.loop(0, NCH, step=2)
    def _chunk(r):
        for b in range(2):
            rr = r + b
            src_b, dst_b, ae_b, an_b, rbf_b, gsem_b, psem_b, asem_b = bufs[b]
            src_o, dst_o, ae_o, an_o, rbf_o, gsem_o, psem_o, asem_o = \
                bufs[1 - b]

            @pl.when(rr + 1 < NCH)
            def _launch_next():
                pltpu.make_async_copy(srcp_hbm.at[wid, rr + 1], src_o,
                                      psem_o).wait()
                pltpu.make_async_copy(dstp_hbm.at[wid, rr + 1], dst_o,
                                      psem_o).wait()
                pltpu.make_async_copy(aexp_hbm.at[wid, rr + 1], ae_o,
                                      psem_o).wait()
                pltpu.async_copy(whbf_hbm.at[src_o], rbf_o, gsem_o)

            @pl.when(rr >= 2)
            def _drain_an():
                pltpu.make_async_copy(an_b, an_hbm.at[wid, rr], asem_b).wait()

            for g in range(8):
                sl = pl.ds(g * 16, 16)
                asg = plsc.load_gather(asum_v, [dst_b[sl]])
                an_b[sl] = ae_b[sl] / (asg + 1e-9)
            pltpu.async_copy(an_b, an_hbm.at[wid, rr], asem_b)

            pltpu.make_async_copy(whbf_hbm.at[src_b], rbf_b, gsem_b).wait()

            @pl.loop(0, CH, unroll=2)
            def _scale(e):
                aeb = plsc.load_gather(ae_b, [jnp.full((16,), e, _i32)])
                for g in range(4):
                    x = rbf_b[e, pl.ds(g * 16, 16)]
                    lo = plsc.bitcast(lax.shift_left(x, 16), _f32)
                    hi = plsc.bitcast(
                        lax.bitwise_and(x, jnp.int32(-65536)), _f32)
                    rows_f[e, pl.ds(g * 32, 16)] = lo * aeb
                    rows_f[e, pl.ds(g * 32 + 16, 16)] = hi * aeb

            pltpu.sync_copy(rows_f, acc_sh.at[dst_b], add=True)

            @pl.when(rr + 2 < NCH)
            def _prefetch():
                pltpu.async_copy(srcp_hbm.at[wid, rr + 2], src_b, psem_b)
                pltpu.async_copy(dstp_hbm.at[wid, rr + 2], dst_b, psem_b)
                pltpu.async_copy(aexp_hbm.at[wid, rr + 2], ae_b, psem_b)

    pltpu.make_async_copy(an0, an_hbm.at[wid, NCH - 2], asem0).wait()
    pltpu.make_async_copy(an1, an_hbm.at[wid, NCH - 1], asem1).wait()
    plsc.subcore_barrier()

    half = CH // 2
    nko = ROWS_PER_TILE // half
    for k in range(nko):
        hsl = pl.ds((k % 2) * half, half)
        wsem = gsem0 if k % 2 == 0 else gsem1
        base = sid * ROWS_PER_TILE + k * half
        if k >= 2:
            pltpu.make_async_copy(
                rows_f.at[pl.ds(((k - 2) % 2) * half, half)],
                outp_hbm.at[cid, pl.ds(sid * ROWS_PER_TILE
                                       + (k - 2) * half, half)], wsem).wait()
        pltpu.sync_copy(acc_sh.at[pl.ds(base, half)], rows_f.at[hsl])
        pltpu.async_copy(rows_f.at[hsl], outp_hbm.at[cid, pl.ds(base, half)],
                         wsem)
    for k in (nko - 2, nko - 1):
        wsem = gsem0 if k % 2 == 0 else gsem1
        pltpu.make_async_copy(
            rows_f.at[pl.ds((k % 2) * half, half)],
            outp_hbm.at[cid, pl.ds(sid * ROWS_PER_TILE + k * half, half)],
            wsem).wait()


def _pass2(srcp, dstp, aexp, hist, whbf):
    mesh = plsc.VectorSubcoreMesh(core_axis_name="c", subcore_axis_name="s")
    fn = pl.kernel(
        _pass2_body,
        out_type=(
            jax.ShapeDtypeStruct((NT, NCH, CH), _f32),
            jax.ShapeDtypeStruct((NC, NPAD, DIM), _f32),
        ),
        mesh=mesh,
        scratch_types=[
            pltpu.VMEM((CH,), _i32),
            pltpu.VMEM((CH,), _i32),
            pltpu.VMEM((CH,), _i32),
            pltpu.VMEM((CH,), _i32),
            pltpu.VMEM((CH,), _f32),
            pltpu.VMEM((CH,), _f32),
            pltpu.VMEM((CH,), _f32),
            pltpu.VMEM((CH,), _f32),
            pltpu.VMEM((NPAD,), _f32),
            pltpu.VMEM((NPAD // 8,), _f32),
            pltpu.VMEM((CH, DIM // 2), _i32),
            pltpu.VMEM((CH, DIM // 2), _i32),
            pltpu.VMEM((CH, DIM), _f32),
            pltpu.VMEM_SHARED((NPAD, DIM), _f32),
            pltpu.SemaphoreType.DMA,
            pltpu.SemaphoreType.DMA,
            pltpu.SemaphoreType.DMA,
            pltpu.SemaphoreType.DMA,
            pltpu.SemaphoreType.DMA,
            pltpu.SemaphoreType.DMA,
        ],
        compiler_params=pltpu.CompilerParams(needs_layout_passes=False,
                                             use_tc_tiling_on_sc=False),
    )
    return fn(srcp, dstp, aexp, hist, whbf)


def _fin_body(p_ref, a_ref, o_ref):
    x = (p_ref[0] + p_ref[1]) / (a_ref[...] + 1e-9)
    o_ref[...] = jnp.where(x > 0, x, jnp.exp(x) - 1.0)


def _fin(outp, asum2):
    blk = 1000
    grid = N_NODES // blk
    return pl.pallas_call(
        _fin_body,
        grid=(grid,),
        in_specs=[
            pl.BlockSpec((NC, blk, DIM), lambda i: (0, i, 0)),
            pl.BlockSpec((blk, 1), lambda i: (i, 0)),
        ],
        out_specs=pl.BlockSpec((blk, DIM), lambda i: (i, 0)),
        out_shape=jax.ShapeDtypeStruct((N_NODES, DIM), _f32),
    )(outp, asum2)


def kernel(h, edge_index, W, a_w):
    a12 = a_w.reshape(2, DIM)

    rot = jnp.arange(EPAD - N_EDGES, dtype=_i32) % (NPAD - N_NODES)
    srcp = jnp.concatenate([edge_index[0], rot]).reshape(NT, NCH, CH)
    dstp = jnp.concatenate([edge_index[1], N_NODES + rot]).reshape(
        NT, NCH, CH)

    sd = _sdprep(h, W, a12)
    wh = _prep(h, W)
    whbf = lax.bitcast_convert_type(
        wh.astype(jnp.bfloat16)[:, _PERM].reshape(N_NODES, DIM // 2, 2),
        jnp.int32)
    aexp, hist = _pass1(sd, srcp, dstp)
    anorm, outp = _pass2(srcp, dstp, aexp, hist, whbf)
    asum2 = (hist[0, :N_NODES] + hist[1, :N_NODES])[:, None]
    out = _fin(outp, asum2)
    return (out, anorm.reshape(EPAD)[:N_EDGES])


# speedup vs baseline: 1.7834x; 1.7832x over previous
import functools

import jax
import jax.numpy as jnp
import numpy as np
from jax import lax
from jax.experimental import pallas as pl
from jax.experimental.pallas import tpu as pltpu
from jax.experimental.pallas import tpu_sc as plsc

N_NODES = 10000
N_EDGES = 320000
DIM = 128

NPAD = 10240
PAD_IDX = 10200
NC = 2
NS = 16
NT = NC * NS
CH = 128
NCH = 80
EPT = NCH * CH
EPAD = NT * EPT
ROWS_PER_TILE = NPAD // NS

_f32 = jnp.float32
_i32 = jnp.int32

_PERM = np.arange(DIM).reshape(DIM // 32, 2, 16).transpose(0, 2, 1).reshape(
    DIM)


def _prep_body(h_ref, w_ref, wh_ref):
    hb = h_ref[...]
    wm = w_ref[...]
    wh_ref[...] = lax.dot_general(hb, wm, (((1,), (1,)), ((), ())),
                                  preferred_element_type=_f32)


def _prep(h, W):
    blk = 1000
    grid = N_NODES // blk
    return pl.pallas_call(
        _prep_body,
        grid=(grid,),
        in_specs=[
            pl.BlockSpec((blk, DIM), lambda i: (i, 0)),
            pl.BlockSpec((DIM, DIM), lambda i: (0, 0)),
        ],
        out_specs=pl.BlockSpec((blk, DIM), lambda i: (i, 0)),
        out_shape=jax.ShapeDtypeStruct((N_NODES, DIM), _f32),
    )(h, W)


def _sdprep_body(h_ref, w_ref, a_ref, sd_ref):
    aw = lax.dot_general(a_ref[...], w_ref[...], (((1,), (0,)), ((), ())),
                         preferred_element_type=_f32)
    sd = lax.dot_general(aw, h_ref[...], (((1,), (1,)), ((), ())),
                         preferred_element_type=_f32)
    sd_ref[...] = jnp.concatenate(
        [sd, jnp.zeros((2, NPAD - N_NODES), _f32)], axis=1)


def _sdprep(h, W, a12):
    return pl.pallas_call(
        _sdprep_body,
        out_shape=jax.ShapeDtypeStruct((2, NPAD), _f32),
    )(h, W, a12)


def _pass1_body(sd_hbm, srcp_hbm, dstp_hbm,
                aexp_hbm, hist_hbm,
                s_v, d_v, src_v, dst_v, aexp_v, zero_v, hist_sh, ldsem):
    cid = lax.axis_index("c")
    sid = lax.axis_index("s")
    wid = cid * NS + sid

    c1 = pltpu.async_copy(sd_hbm.at[0], s_v, ldsem)
    c2 = pltpu.async_copy(sd_hbm.at[1], d_v, ldsem)
    c4 = pltpu.async_copy(srcp_hbm.at[wid], src_v, ldsem)
    c5 = pltpu.async_copy(dstp_hbm.at[wid], dst_v, ldsem)

    for k in range(ROWS_PER_TILE // 16):
        zero_v[pl.ds(k * 16, 16)] = jnp.zeros((16,), _f32)
    pltpu.sync_copy(zero_v, hist_sh.at[pl.ds(sid * ROWS_PER_TILE,
                                             ROWS_PER_TILE)])
    for c in (c1, c2, c4, c5):
        c.wait()
    plsc.subcore_barrier()

    def _maxstep(i, carry):
        ms, md = carry
        sl = pl.ds(i * 16, 16)
        return (jnp.maximum(ms, s_v[sl]), jnp.maximum(md, d_v[sl]))

    ms0 = jnp.full((16,), -jnp.inf, _f32)
    ms, md = lax.fori_loop(0, NPAD // 16, _maxstep, (ms0, ms0),
                           unroll=2)
    mvec = jnp.full((16,), jnp.maximum(
        lax.reduce_max(ms, (0,)) + lax.reduce_max(md, (0,)), 0.0), _f32)

    @pl.loop(0, NCH, unroll=2)
    def _compute(r):
        for g in range(8):
            sl = pl.ds(g * 16, 16)
            si = src_v[r, sl]
            di = dst_v[r, sl]
            sg = plsc.load_gather(s_v, [si])
            dg = plsc.load_gather(d_v, [di])
            al = sg + dg
            al = jnp.where(al > 0, al, al * 0.2)
            aexp_v[r, sl] = jnp.exp(al - mvec)

    @pl.loop(0, NCH)
    def _scatter(r):
        pltpu.async_copy(aexp_v.at[r], hist_sh.at[dst_v.at[r]], ldsem,
                         add=True)

    @pl.loop(0, NCH)
    def _drain(r):
        pltpu.make_async_copy(aexp_v.at[r], hist_sh.at[dst_v.at[r]],
                              ldsem).wait()

    plsc.subcore_barrier()
    pltpu.sync_copy(aexp_v, aexp_hbm.at[wid])

    @pl.when(sid == 0)
    def _dump():
        pltpu.sync_copy(hist_sh, s_v)
        pltpu.sync_copy(s_v, hist_hbm.at[cid])


def _pass1(sd, srcp, dstp):
    mesh = plsc.VectorSubcoreMesh(core_axis_name="c", subcore_axis_name="s")
    fn = pl.kernel(
        _pass1_body,
        out_type=(
            jax.ShapeDtypeStruct((NT, NCH, CH), _f32),
            jax.ShapeDtypeStruct((NC, NPAD), _f32),
        ),
        mesh=mesh,
        scratch_types=[
            pltpu.VMEM((NPAD,), _f32),
            pltpu.VMEM((NPAD,), _f32),
            pltpu.VMEM((NCH, CH), _i32),
            pltpu.VMEM((NCH, CH), _i32),
            pltpu.VMEM((NCH, CH), _f32),
            pltpu.VMEM((ROWS_PER_TILE,), _f32),
            pltpu.VMEM_SHARED((NPAD,), _f32),
            pltpu.SemaphoreType.DMA,
        ],
        compiler_params=pltpu.CompilerParams(needs_layout_passes=False),
    )
    return fn(sd, srcp, dstp)


def _pass2_body(srcp_hbm, dstp_hbm, aexp_hbm, hist_hbm, wh_hbm,
                an_hbm, outp_hbm,
                src0, src1, dst0, dst1, sdst0, sdst1, ae0, ae1, an0, an1,
                asum_v, tmp_v, rows0, rows1, acc_sh,
                gsem0, gsem1, psem0, psem1, ssem0, ssem1, asem0, asem1):
    cid = lax.axis_index("c")
    sid = lax.axis_index("s")
    wid = cid * NS + sid

    bufs = [(src0, dst0, sdst0, ae0, an0, rows0, gsem0, psem0, ssem0, asem0),
            (src1, dst1, sdst1, ae1, an1, rows1, gsem1, psem1, ssem1, asem1)]

    pltpu.sync_copy(hist_hbm.at[0], asum_v)
    for k in range(8):
        csl = pl.ds(k * (NPAD // 8), NPAD // 8)
        pltpu.sync_copy(hist_hbm.at[1, csl], tmp_v)

        @pl.loop(0, NPAD // 8 // 16)
        def _acc(i):
            sl = pl.ds(k * (NPAD // 8) + i * 16, 16)
            tsl = pl.ds(i * 16, 16)
            asum_v[sl] = asum_v[sl] + tmp_v[tsl]

    @pl.loop(0, CH)
    def _zero(r):
        for g in range(8):
            rows0[r, pl.ds(g * 16, 16)] = jnp.zeros((16,), _f32)

    for k in range(ROWS_PER_TILE // CH):
        pltpu.sync_copy(rows0,
                        acc_sh.at[pl.ds(sid * ROWS_PER_TILE + k * CH, CH)])
    plsc.subcore_barrier()

    pltpu.sync_copy(srcp_hbm.at[wid, 0], src0)
    pltpu.sync_copy(dstp_hbm.at[wid, 0], dst0)
    pltpu.sync_copy(aexp_hbm.at[wid, 0], ae0)
    pltpu.async_copy(wh_hbm.at[src0], rows0, gsem0)
    pltpu.async_copy(srcp_hbm.at[wid, 1], src1, psem1)
    pltpu.async_copy(dstp_hbm.at[wid, 1], dst1, psem1)
    pltpu.async_copy(aexp_hbm.at[wid, 1], ae1, psem1)

    @pl.loop(0, NCH, step=2)
    def _chunk(r):
        for b in range(2):
            rr = r + b
            src_b, dst_b, sdst_b, ae_b, an_b, rows_b, gsem_b, psem_b, \
                ssem_b, asem_b = bufs[b]
            src_o, dst_o, sdst_o, ae_o, an_o, rows_o, gsem_o, psem_o, \
                ssem_o, asem_o = bufs[1 - b]

            @pl.when(rr + 1 < NCH)
            def _launch_next():
                pltpu.make_async_copy(srcp_hbm.at[wid, rr + 1], src_o,
                                      psem_o).wait()
                pltpu.make_async_copy(dstp_hbm.at[wid, rr + 1], dst_o,
                                      psem_o).wait()
                pltpu.make_async_copy(aexp_hbm.at[wid, rr + 1], ae_o,
                                      psem_o).wait()

                @pl.when(rr >= 1)
                def _drain_scatter():
                    pltpu.make_async_copy(rows_o, acc_sh.at[sdst_o],
                                          ssem_o).wait()

                pltpu.async_copy(wh_hbm.at[src_o], rows_o, gsem_o)

            @pl.when(rr >= 2)
            def _drain_an():
                pltpu.make_async_copy(an_b, an_hbm.at[wid, rr], asem_b).wait()

            for g in range(8):
                sl = pl.ds(g * 16, 16)
                asg = plsc.load_gather(asum_v, [dst_b[sl]])
                an_b[sl] = ae_b[sl] / (asg + 1e-9)
            pltpu.async_copy(an_b, an_hbm.at[wid, rr], asem_b)

            pltpu.make_async_copy(wh_hbm.at[src_b], rows_b, gsem_b).wait()

            @pl.loop(0, CH, unroll=2)
            def _scale(e):
                aeb = plsc.load_gather(ae_b, [jnp.full((16,), e, _i32)])
                for g in range(8):
                    sl = pl.ds(g * 16, 16)
                    rows_b[e, sl] = rows_b[e, sl] * aeb

            for g in range(8):
                sl = pl.ds(g * 16, 16)
                sdst_b[sl] = dst_b[sl]
            pltpu.async_copy(rows_b, acc_sh.at[sdst_b], ssem_b, add=True)

            @pl.when(rr + 2 < NCH)
            def _prefetch():
                pltpu.async_copy(srcp_hbm.at[wid, rr + 2], src_b, psem_b)
                pltpu.async_copy(dstp_hbm.at[wid, rr + 2], dst_b, psem_b)
                pltpu.async_copy(aexp_hbm.at[wid, rr + 2], ae_b, psem_b)

    pltpu.make_async_copy(rows0, acc_sh.at[sdst0], ssem0).wait()
    pltpu.make_async_copy(rows1, acc_sh.at[sdst1], ssem1).wait()
    pltpu.make_async_copy(an0, an_hbm.at[wid, NCH - 2], asem0).wait()
    pltpu.make_async_copy(an1, an_hbm.at[wid, NCH - 1], asem1).wait()
    plsc.subcore_barrier()

    for k in range(ROWS_PER_TILE // CH):
        rb, wsem = (rows0, gsem0) if k % 2 == 0 else (rows1, gsem1)
        base = sid * ROWS_PER_TILE + k * CH
        if k >= 2:
            pltpu.make_async_copy(
                rb, outp_hbm.at[cid, pl.ds(sid * ROWS_PER_TILE
                                           + (k - 2) * CH, CH)], wsem).wait()
        pltpu.sync_copy(acc_sh.at[pl.ds(base, CH)], rb)
        pltpu.async_copy(rb, outp_hbm.at[cid, pl.ds(base, CH)], wsem)
    for k in (ROWS_PER_TILE // CH - 2, ROWS_PER_TILE // CH - 1):
        rb, wsem = (rows0, gsem0) if k % 2 == 0 else (rows1, gsem1)
        pltpu.make_async_copy(
            rb, outp_hbm.at[cid, pl.ds(sid * ROWS_PER_TILE + k * CH, CH)],
            wsem).wait()


def _pass2(srcp, dstp, aexp, hist, wh):
    mesh = plsc.VectorSubcoreMesh(core_axis_name="c", subcore_axis_name="s")
    fn = pl.kernel(
        _pass2_body,
        out_type=(
            jax.ShapeDtypeStruct((NT, NCH, CH), _f32),
            jax.ShapeDtypeStruct((NC, NPAD, DIM), _f32),
        ),
        mesh=mesh,
        scratch_types=[
            pltpu.VMEM((CH,), _i32),
            pltpu.VMEM((CH,), _i32),
            pltpu.VMEM((CH,), _i32),
            pltpu.VMEM((CH,), _i32),
            pltpu.VMEM((CH,), _i32),
            pltpu.VMEM((CH,), _i32),
            pltpu.VMEM((CH,), _f32),
            pltpu.VMEM((CH,), _f32),
            pltpu.VMEM((CH,), _f32),
            pltpu.VMEM((CH,), _f32),
            pltpu.VMEM((NPAD,), _f32),
            pltpu.VMEM((NPAD // 8,), _f32),
            pltpu.VMEM((CH, DIM), _f32),
            pltpu.VMEM((CH, DIM), _f32),
            pltpu.VMEM_SHARED((NPAD, DIM), _f32),
            pltpu.SemaphoreType.DMA,
            pltpu.SemaphoreType.DMA,
            pltpu.SemaphoreType.DMA,
            pltpu.SemaphoreType.DMA,
            pltpu.SemaphoreType.DMA,
            pltpu.SemaphoreType.DMA,
            pltpu.SemaphoreType.DMA,
            pltpu.SemaphoreType.DMA,
        ],
        compiler_params=pltpu.CompilerParams(needs_layout_passes=False),
    )
    return fn(srcp, dstp, aexp, hist, wh)


def _fin_body(p_ref, a_ref, o_ref):
    x = (p_ref[0] + p_ref[1]) / (a_ref[...] + 1e-9)
    o_ref[...] = jnp.where(x > 0, x, jnp.exp(x) - 1.0)


def _fin(outp, asum2):
    blk = 1000
    grid = N_NODES // blk
    return pl.pallas_call(
        _fin_body,
        grid=(grid,),
        in_specs=[
            pl.BlockSpec((NC, blk, DIM), lambda i: (0, i, 0)),
            pl.BlockSpec((blk, 1), lambda i: (i, 0)),
        ],
        out_specs=pl.BlockSpec((blk, DIM), lambda i: (i, 0)),
        out_shape=jax.ShapeDtypeStruct((N_NODES, DIM), _f32),
    )(outp, asum2)


def kernel(h, edge_index, W, a_w):
    a12 = a_w.reshape(2, DIM)

    rot = jnp.arange(EPAD - N_EDGES, dtype=_i32) % (NPAD - N_NODES)
    srcp = jnp.concatenate([edge_index[0], rot]).reshape(NT, NCH, CH)
    dstp = jnp.concatenate([edge_index[1], N_NODES + rot]).reshape(
        NT, NCH, CH)

    sd = _sdprep(h, W, a12)
    wh = _prep(h, W)
    aexp, hist = _pass1(sd, srcp, dstp)
    anorm, outp = _pass2(srcp, dstp, aexp, hist, wh)
    asum2 = (hist[0, :N_NODES] + hist[1, :N_NODES])[:, None]
    out = _fin(outp, asum2)
    return (out, anorm.reshape(EPAD)[:N_EDGES])
